# Initial kernel scaffold; baseline (speedup 1.0000x reference)
import jax
import jax.numpy as jnp
from jax.experimental import pallas as pl


def kernel(word_ids, position_ids, adj, edge_types, cls_node, tok_table, pos_table, edge_table, Wq, Wk, Wv, Wo, ln1_g, ln1_b, W1, b1, W2, b2, ln2_g, ln2_b, Wc, bc):
    raise NotImplementedError("write your pallas kernel here")


# speedup vs baseline: 1035.8670x; 1035.8670x over previous
import functools

import jax
import jax.numpy as jnp
from jax.experimental import pallas as pl
from jax.experimental.pallas import tpu as pltpu

N = 2048
D = 128
H = 8
DH = 16
L = 2
FF = 512
NET = 17
NCLASS = 5
B = 16

BN = 256
NQB = N // BN


def _ln(x, g, b, eps=1e-5):
    m = jnp.mean(x, axis=-1, keepdims=True)
    v = jnp.mean((x - m) ** 2, axis=-1, keepdims=True)
    return (x - m) / jnp.sqrt(v + eps) * g + b


def _gat_kernel(h0_ref, adj_ref, et_ref, ek_ref, wq_ref, wk_ref, wv_ref,
                wo_ref, ln1g_ref, ln1b_ref, w1_ref, b1_ref, w2_ref, b2_ref,
                ln2g_ref, ln2b_ref, cls_ref, wc_ref, bc_ref,
                out_ref, h_scr, k_scr, v_scr):
    l = pl.program_id(0)
    i = pl.program_id(1)
    scale = jnp.float32(1.0 / (DH ** 0.5))

    @pl.when((l == 0) & (i == 0))
    def _():
        h_scr[...] = h0_ref[...]

    @pl.when(i == 0)
    def _():
        hf = h_scr[...]
        k_scr[...] = jnp.dot(hf, wk_ref[...].reshape(D, D),
                             preferred_element_type=jnp.float32)
        v_scr[...] = jnp.dot(hf, wv_ref[...].reshape(D, D),
                             preferred_element_type=jnp.float32)

    hb = h_scr[pl.ds(i * BN, BN), :]
    q = jnp.dot(hb, wq_ref[...].reshape(D, D),
                preferred_element_type=jnp.float32)
    k = k_scr[...]
    v = v_scr[...]
    ek = ek_ref[...]
    adjb = adj_ref[...]
    etb = et_ref[...]

    ctx_parts = []
    for hh in range(H):
        sl = slice(hh * DH, (hh + 1) * DH)
        qh = q[:, sl]
        kh = k[:, sl]
        s = jax.lax.dot_general(
            qh, kh, (((1,), (1,)), ((), ())),
            preferred_element_type=jnp.float32) * scale
        qe = jax.lax.dot_general(
            qh, ek[:, sl], (((1,), (1,)), ((), ())),
            preferred_element_type=jnp.float32) * scale
        for t in range(NET):
            s = s + jnp.where(etb == t, qe[:, t:t + 1], 0.0)
        s = jnp.where(adjb > 0, s, jnp.float32(-1e9))
        m = jnp.max(s, axis=1, keepdims=True)
        p = jnp.exp(s - m)
        p = p / jnp.sum(p, axis=1, keepdims=True)
        ctx_parts.append(jax.lax.dot_general(
            p, v[:, sl], (((1,), (0,)), ((), ())),
            preferred_element_type=jnp.float32))
    ctx = jnp.concatenate(ctx_parts, axis=1)

    h1 = hb + jnp.dot(ctx, wo_ref[...].reshape(D, D),
                      preferred_element_type=jnp.float32)
    h1 = _ln(h1, ln1g_ref[...].reshape(1, D), ln1b_ref[...].reshape(1, D))
    ffv = jnp.maximum(
        jnp.dot(h1, w1_ref[...].reshape(D, FF),
                preferred_element_type=jnp.float32)
        + b1_ref[...].reshape(1, FF), 0.0)
    ffv = jnp.dot(ffv, w2_ref[...].reshape(FF, D),
                  preferred_element_type=jnp.float32) + b2_ref[...].reshape(1, D)
    hn = _ln(h1 + ffv, ln2g_ref[...].reshape(1, D), ln2b_ref[...].reshape(1, D))
    h_scr[pl.ds(i * BN, BN), :] = hn

    @pl.when((l == L - 1) & (i == NQB - 1))
    def _():
        hf = h_scr[...]
        idx = cls_ref[...]
        iota = jax.lax.broadcasted_iota(jnp.int32, (B, N), 1)
        oh = (iota == idx).astype(jnp.float32)
        clsh = jnp.dot(oh, hf, preferred_element_type=jnp.float32)
        out_ref[...] = jnp.dot(clsh, wc_ref[...],
                               preferred_element_type=jnp.float32) + bc_ref[...]


@jax.jit
def _gat_forward(h0, adj, edge_types, edge_table, Wq, Wk, Wv, Wo,
                 ln1_g, ln1_b, W1, b1, W2, b2, ln2_g, ln2_b,
                 cls_node2d, Wc, bc2d):
    grid = (L, NQB)
    full = lambda shape: pl.BlockSpec(shape, lambda l, i: (0,) * len(shape))
    per_l = lambda shape: pl.BlockSpec((1,) + shape, lambda l, i: (l,) + (0,) * len(shape))
    row_block = pl.BlockSpec((BN, N), lambda l, i: (i, 0))
    return pl.pallas_call(
        _gat_kernel,
        grid=grid,
        in_specs=[
            full((N, D)),
            row_block,
            row_block,
            full((NET, D)),
            per_l((D, D)),
            per_l((D, D)),
            per_l((D, D)),
            per_l((D, D)),
            per_l((1, D)),
            per_l((1, D)),
            per_l((D, FF)),
            per_l((1, FF)),
            per_l((FF, D)),
            per_l((1, D)),
            per_l((1, D)),
            per_l((1, D)),
            full((B, 1)),
            full((D, NCLASS)),
            full((1, NCLASS)),
        ],
        out_specs=pl.BlockSpec((B, NCLASS), lambda l, i: (0, 0)),
        out_shape=jax.ShapeDtypeStruct((B, NCLASS), jnp.float32),
        scratch_shapes=[
            pltpu.VMEM((N, D), jnp.float32),
            pltpu.VMEM((N, D), jnp.float32),
            pltpu.VMEM((N, D), jnp.float32),
        ],
    )(h0, adj, edge_types, edge_table, Wq, Wk, Wv, Wo,
      ln1_g, ln1_b, W1, b1, W2, b2, ln2_g, ln2_b,
      cls_node2d, Wc, bc2d)


def kernel(word_ids, position_ids, adj, edge_types, cls_node, tok_table,
           pos_table, edge_table, Wq, Wk, Wv, Wo, ln1_g, ln1_b, W1, b1,
           W2, b2, ln2_g, ln2_b, Wc, bc):
    h0 = jnp.take(tok_table, word_ids, axis=0) + \
        jnp.take(pos_table, position_ids, axis=0)
    return _gat_forward(
        h0, adj, edge_types, edge_table, Wq, Wk, Wv, Wo,
        ln1_g.reshape(L, 1, D), ln1_b.reshape(L, 1, D),
        W1, b1.reshape(L, 1, FF), W2, b2.reshape(L, 1, D),
        ln2_g.reshape(L, 1, D), ln2_b.reshape(L, 1, D),
        cls_node.astype(jnp.int32).reshape(B, 1), Wc, bc.reshape(1, NCLASS))


# speedup vs baseline: 1411.1130x; 1.3623x over previous
import functools

import jax
import jax.numpy as jnp
from jax.experimental import pallas as pl
from jax.experimental.pallas import tpu as pltpu

N = 2048
D = 128
H = 8
DH = 16
L = 2
FF = 512
NET = 17
NCLASS = 5
B = 16

BN = 128
NQB = N // BN


def _ln(x, g, b, eps=1e-5):
    m = jnp.mean(x, axis=-1, keepdims=True)
    v = jnp.mean((x - m) ** 2, axis=-1, keepdims=True)
    return (x - m) / jnp.sqrt(v + eps) * g + b


def _gat_kernel(h0_ref, adj_ref, et_ref, ek_ref, wq_ref, wk_ref, wv_ref,
                wo_ref, ln1g_ref, ln1b_ref, w1_ref, b1_ref, w2_ref, b2_ref,
                ln2g_ref, ln2b_ref, cls_ref, wc_ref, bc_ref,
                out_ref, h_scr, k_scr, v_scr):
    l = pl.program_id(0)
    i = pl.program_id(1)
    scale = jnp.float32(1.0 / (DH ** 0.5))

    @pl.when((l == 0) & (i == 0))
    def _():
        h_scr[...] = h0_ref[...]

    @pl.when(i == 0)
    def _():
        hf = h_scr[...]
        k_scr[...] = jnp.dot(hf, wk_ref[...].reshape(D, D),
                             preferred_element_type=jnp.float32)
        v_scr[...] = jnp.dot(hf, wv_ref[...].reshape(D, D),
                             preferred_element_type=jnp.float32)

    hb = h_scr[pl.ds(i * BN, BN), :]
    q = jnp.dot(hb, wq_ref[...].reshape(D, D),
                preferred_element_type=jnp.float32)
    k = k_scr[...]
    v = v_scr[...]
    ek = ek_ref[...]
    adjb = adj_ref[...]
    etb = et_ref[...]

    amask = jnp.where(adjb > 0, 0.0, jnp.float32(-1e9))
    tmasks = [(etb == t).astype(jnp.float32) for t in range(NET)]

    ctx_parts = []
    for hh in range(H):
        sl = slice(hh * DH, (hh + 1) * DH)
        qh = q[:, sl]
        kh = k[:, sl]
        s = jax.lax.dot_general(
            qh, kh, (((1,), (1,)), ((), ())),
            preferred_element_type=jnp.float32) * scale
        qe = jax.lax.dot_general(
            qh, ek[:, sl], (((1,), (1,)), ((), ())),
            preferred_element_type=jnp.float32) * scale
        for t in range(NET):
            s = s + tmasks[t] * qe[:, t:t + 1]
        s = s + amask
        m = jnp.max(s, axis=1, keepdims=True)
        p = jnp.exp(s - m)
        denom = jnp.sum(p, axis=1, keepdims=True)
        ctx_parts.append(jax.lax.dot_general(
            p, v[:, sl], (((1,), (0,)), ((), ())),
            preferred_element_type=jnp.float32) / denom)
    ctx = jnp.concatenate(ctx_parts, axis=1)

    h1 = hb + jnp.dot(ctx, wo_ref[...].reshape(D, D),
                      preferred_element_type=jnp.float32)
    h1 = _ln(h1, ln1g_ref[...].reshape(1, D), ln1b_ref[...].reshape(1, D))
    ffv = jnp.maximum(
        jnp.dot(h1, w1_ref[...].reshape(D, FF),
                preferred_element_type=jnp.float32)
        + b1_ref[...].reshape(1, FF), 0.0)
    ffv = jnp.dot(ffv, w2_ref[...].reshape(FF, D),
                  preferred_element_type=jnp.float32) + b2_ref[...].reshape(1, D)
    hn = _ln(h1 + ffv, ln2g_ref[...].reshape(1, D), ln2b_ref[...].reshape(1, D))
    h_scr[pl.ds(i * BN, BN), :] = hn

    @pl.when((l == L - 1) & (i == NQB - 1))
    def _():
        hf = h_scr[...]
        idx = cls_ref[...]
        iota = jax.lax.broadcasted_iota(jnp.int32, (B, N), 1)
        oh = (iota == idx).astype(jnp.float32)
        clsh = jnp.dot(oh, hf, preferred_element_type=jnp.float32)
        out_ref[...] = jnp.dot(clsh, wc_ref[...],
                               preferred_element_type=jnp.float32) + bc_ref[...]


@jax.jit
def _gat_forward(h0, adj, edge_types, edge_table, Wq, Wk, Wv, Wo,
                 ln1_g, ln1_b, W1, b1, W2, b2, ln2_g, ln2_b,
                 cls_node2d, Wc, bc2d):
    grid = (L, NQB)
    full = lambda shape: pl.BlockSpec(shape, lambda l, i: (0,) * len(shape))
    per_l = lambda shape: pl.BlockSpec((1,) + shape, lambda l, i: (l,) + (0,) * len(shape))
    row_block = pl.BlockSpec((BN, N), lambda l, i: (i, 0))
    return pl.pallas_call(
        _gat_kernel,
        grid=grid,
        in_specs=[
            full((N, D)),
            row_block,
            row_block,
            full((NET, D)),
            per_l((D, D)),
            per_l((D, D)),
            per_l((D, D)),
            per_l((D, D)),
            per_l((1, D)),
            per_l((1, D)),
            per_l((D, FF)),
            per_l((1, FF)),
            per_l((FF, D)),
            per_l((1, D)),
            per_l((1, D)),
            per_l((1, D)),
            full((B, 1)),
            full((D, NCLASS)),
            full((1, NCLASS)),
        ],
        out_specs=pl.BlockSpec((B, NCLASS), lambda l, i: (0, 0)),
        out_shape=jax.ShapeDtypeStruct((B, NCLASS), jnp.float32),
        scratch_shapes=[
            pltpu.VMEM((N, D), jnp.float32),
            pltpu.VMEM((N, D), jnp.float32),
            pltpu.VMEM((N, D), jnp.float32),
        ],
    )(h0, adj, edge_types, edge_table, Wq, Wk, Wv, Wo,
      ln1_g, ln1_b, W1, b1, W2, b2, ln2_g, ln2_b,
      cls_node2d, Wc, bc2d)


def kernel(word_ids, position_ids, adj, edge_types, cls_node, tok_table,
           pos_table, edge_table, Wq, Wk, Wv, Wo, ln1_g, ln1_b, W1, b1,
           W2, b2, ln2_g, ln2_b, Wc, bc):
    h0 = jnp.take(tok_table, word_ids, axis=0) + \
        jnp.take(pos_table, position_ids, axis=0)
    return _gat_forward(
        h0, adj, edge_types, edge_table, Wq, Wk, Wv, Wo,
        ln1_g.reshape(L, 1, D), ln1_b.reshape(L, 1, D),
        W1, b1.reshape(L, 1, FF), W2, b2.reshape(L, 1, D),
        ln2_g.reshape(L, 1, D), ln2_b.reshape(L, 1, D),
        cls_node.astype(jnp.int32).reshape(B, 1), Wc, bc.reshape(1, NCLASS))


# speedup vs baseline: 1433.1354x; 1.0156x over previous
import functools

import jax
import jax.numpy as jnp
from jax import lax
from jax.experimental import pallas as pl
from jax.experimental.pallas import tpu as pltpu
from jax.experimental.pallas import tpu_sc as plsc

N = 2048
D = 128
H = 8
DH = 16
L = 2
FF = 512
NET = 17
NCLASS = 5
B = 16

BN = 128
NQB = N // BN


_NC, _NS, _LANES = 2, 16, 16
_NW = _NC * _NS
_BPW = N // _NW


def _emb_body(tok_hbm, pos_hbm, wid_hbm, pid_hbm, out_hbm,
              idx_v, rows_a, rows_b, sem):
    wid = lax.axis_index("s") * _NC + lax.axis_index("c")
    base = wid * _BPW
    pltpu.sync_copy(wid_hbm.at[pl.ds(base, _BPW)], idx_v)
    pltpu.async_copy(tok_hbm.at[idx_v], rows_a, sem).wait()
    pltpu.sync_copy(pid_hbm.at[pl.ds(base, _BPW)], idx_v)
    pltpu.async_copy(pos_hbm.at[idx_v], rows_b, sem).wait()

    def body(r, carry):
        for c in range(D // _LANES):
            sl = pl.ds(c * _LANES, _LANES)
            rows_a[r, sl] = rows_a[r, sl] + rows_b[r, sl]
        return carry

    lax.fori_loop(0, _BPW, body, 0)
    pltpu.sync_copy(rows_a, out_hbm.at[pl.ds(base, _BPW)])


def _embed_sc(tok_table, pos_table, word_ids, position_ids):
    k = functools.partial(
        pl.kernel, _emb_body,
        mesh=plsc.VectorSubcoreMesh(core_axis_name="c", subcore_axis_name="s"),
        out_type=jax.ShapeDtypeStruct((N, D), jnp.float32),
        scratch_types=[
            pltpu.VMEM((_BPW,), jnp.int32),
            pltpu.VMEM((_BPW, D), jnp.float32),
            pltpu.VMEM((_BPW, D), jnp.float32),
            pltpu.SemaphoreType.DMA,
        ],
    )()
    return k(tok_table, pos_table, word_ids, position_ids)


def _ln(x, g, b, eps=1e-5):
    m = jnp.mean(x, axis=-1, keepdims=True)
    v = jnp.mean((x - m) ** 2, axis=-1, keepdims=True)
    return (x - m) / jnp.sqrt(v + eps) * g + b


def _gat_kernel(h0_ref, adj_ref, et_ref, ek_ref, wq_ref, wk_ref, wv_ref,
                wo_ref, ln1g_ref, ln1b_ref, w1_ref, b1_ref, w2_ref, b2_ref,
                ln2g_ref, ln2b_ref, cls_ref, wc_ref, bc_ref,
                out_ref, h_scr, k_scr, v_scr):
    l = pl.program_id(0)
    i = pl.program_id(1)
    scale = jnp.float32(1.0 / (DH ** 0.5))

    @pl.when((l == 0) & (i == 0))
    def _():
        h_scr[...] = h0_ref[...]

    @pl.when(i == 0)
    def _():
        hf = h_scr[...]
        k_scr[...] = jnp.dot(hf, wk_ref[...].reshape(D, D),
                             preferred_element_type=jnp.float32)
        v_scr[...] = jnp.dot(hf, wv_ref[...].reshape(D, D),
                             preferred_element_type=jnp.float32)

    hb = h_scr[pl.ds(i * BN, BN), :]
    q = jnp.dot(hb, wq_ref[...].reshape(D, D),
                preferred_element_type=jnp.float32)
    k = k_scr[...]
    v = v_scr[...]
    ek = ek_ref[...]
    adjb = adj_ref[...]
    etb = et_ref[...]

    amask = jnp.where(adjb > 0, 0.0, jnp.float32(-1e9))
    tmasks = [(etb == t).astype(jnp.float32) for t in range(NET)]

    ctx_parts = []
    for hh in range(H):
        sl = slice(hh * DH, (hh + 1) * DH)
        qh = q[:, sl]
        kh = k[:, sl]
        s = jax.lax.dot_general(
            qh, kh, (((1,), (1,)), ((), ())),
            preferred_element_type=jnp.float32) * scale
        qe = jax.lax.dot_general(
            qh, ek[:, sl], (((1,), (1,)), ((), ())),
            preferred_element_type=jnp.float32) * scale
        for t in range(NET):
            s = s + tmasks[t] * qe[:, t:t + 1]
        s = s + amask
        m = jnp.max(s, axis=1, keepdims=True)
        p = jnp.exp(s - m)
        denom = jnp.sum(p, axis=1, keepdims=True)
        ctx_parts.append(jax.lax.dot_general(
            p, v[:, sl], (((1,), (0,)), ((), ())),
            preferred_element_type=jnp.float32) / denom)
    ctx = jnp.concatenate(ctx_parts, axis=1)

    h1 = hb + jnp.dot(ctx, wo_ref[...].reshape(D, D),
                      preferred_element_type=jnp.float32)
    h1 = _ln(h1, ln1g_ref[...].reshape(1, D), ln1b_ref[...].reshape(1, D))
    ffv = jnp.maximum(
        jnp.dot(h1, w1_ref[...].reshape(D, FF),
                preferred_element_type=jnp.float32)
        + b1_ref[...].reshape(1, FF), 0.0)
    ffv = jnp.dot(ffv, w2_ref[...].reshape(FF, D),
                  preferred_element_type=jnp.float32) + b2_ref[...].reshape(1, D)
    hn = _ln(h1 + ffv, ln2g_ref[...].reshape(1, D), ln2b_ref[...].reshape(1, D))
    h_scr[pl.ds(i * BN, BN), :] = hn

    @pl.when((l == L - 1) & (i == NQB - 1))
    def _():
        hf = h_scr[...]
        idx = cls_ref[...]
        iota = jax.lax.broadcasted_iota(jnp.int32, (B, N), 1)
        oh = (iota == idx).astype(jnp.float32)
        clsh = jnp.dot(oh, hf, preferred_element_type=jnp.float32)
        out_ref[...] = jnp.dot(clsh, wc_ref[...],
                               preferred_element_type=jnp.float32) + bc_ref[...]


@jax.jit
def _gat_forward(h0, adj, edge_types, edge_table, Wq, Wk, Wv, Wo,
                 ln1_g, ln1_b, W1, b1, W2, b2, ln2_g, ln2_b,
                 cls_node2d, Wc, bc2d):
    grid = (L, NQB)
    full = lambda shape: pl.BlockSpec(shape, lambda l, i: (0,) * len(shape))
    per_l = lambda shape: pl.BlockSpec((1,) + shape, lambda l, i: (l,) + (0,) * len(shape))
    row_block = pl.BlockSpec((BN, N), lambda l, i: (i, 0))
    return pl.pallas_call(
        _gat_kernel,
        grid=grid,
        in_specs=[
            full((N, D)),
            row_block,
            row_block,
            full((NET, D)),
            per_l((D, D)),
            per_l((D, D)),
            per_l((D, D)),
            per_l((D, D)),
            per_l((1, D)),
            per_l((1, D)),
            per_l((D, FF)),
            per_l((1, FF)),
            per_l((FF, D)),
            per_l((1, D)),
            per_l((1, D)),
            per_l((1, D)),
            full((B, 1)),
            full((D, NCLASS)),
            full((1, NCLASS)),
        ],
        out_specs=pl.BlockSpec((B, NCLASS), lambda l, i: (0, 0)),
        out_shape=jax.ShapeDtypeStruct((B, NCLASS), jnp.float32),
        scratch_shapes=[
            pltpu.VMEM((N, D), jnp.float32),
            pltpu.VMEM((N, D), jnp.float32),
            pltpu.VMEM((N, D), jnp.float32),
        ],
    )(h0, adj, edge_types, edge_table, Wq, Wk, Wv, Wo,
      ln1_g, ln1_b, W1, b1, W2, b2, ln2_g, ln2_b,
      cls_node2d, Wc, bc2d)


def kernel(word_ids, position_ids, adj, edge_types, cls_node, tok_table,
           pos_table, edge_table, Wq, Wk, Wv, Wo, ln1_g, ln1_b, W1, b1,
           W2, b2, ln2_g, ln2_b, Wc, bc):
    h0 = _embed_sc(tok_table, pos_table,
                   word_ids.astype(jnp.int32), position_ids.astype(jnp.int32))
    return _gat_forward(
        h0, adj, edge_types, edge_table, Wq, Wk, Wv, Wo,
        ln1_g.reshape(L, 1, D), ln1_b.reshape(L, 1, D),
        W1, b1.reshape(L, 1, FF), W2, b2.reshape(L, 1, D),
        ln2_g.reshape(L, 1, D), ln2_b.reshape(L, 1, D),
        cls_node.astype(jnp.int32).reshape(B, 1), Wc, bc.reshape(1, NCLASS))


# speedup vs baseline: 1921.3296x; 1.3406x over previous
import functools

import jax
import jax.numpy as jnp
from jax import lax
from jax.experimental import pallas as pl
from jax.experimental.pallas import tpu as pltpu
from jax.experimental.pallas import tpu_sc as plsc

N = 2048
D = 128
H = 8
DH = 16
L = 2
FF = 512
NET = 17
NCLASS = 5
B = 16

BN = 128
NQB = N // BN


_NC, _NS, _LANES = 2, 16, 16
_NW = _NC * _NS
_BPW = N // _NW


def _emb_body(tok_hbm, pos_hbm, wid_hbm, pid_hbm, out_hbm,
              idx_v, rows_a, rows_b, sem):
    wid = lax.axis_index("s") * _NC + lax.axis_index("c")
    base = wid * _BPW
    pltpu.sync_copy(wid_hbm.at[pl.ds(base, _BPW)], idx_v)
    pltpu.async_copy(tok_hbm.at[idx_v], rows_a, sem).wait()
    pltpu.sync_copy(pid_hbm.at[pl.ds(base, _BPW)], idx_v)
    pltpu.async_copy(pos_hbm.at[idx_v], rows_b, sem).wait()

    def body(r, carry):
        for c in range(D // _LANES):
            sl = pl.ds(c * _LANES, _LANES)
            rows_a[r, sl] = rows_a[r, sl] + rows_b[r, sl]
        return carry

    lax.fori_loop(0, _BPW, body, 0)
    pltpu.sync_copy(rows_a, out_hbm.at[pl.ds(base, _BPW)])


def _embed_sc(tok_table, pos_table, word_ids, position_ids):
    k = functools.partial(
        pl.kernel, _emb_body,
        mesh=plsc.VectorSubcoreMesh(core_axis_name="c", subcore_axis_name="s"),
        out_type=jax.ShapeDtypeStruct((N, D), jnp.float32),
        scratch_types=[
            pltpu.VMEM((_BPW,), jnp.int32),
            pltpu.VMEM((_BPW, D), jnp.float32),
            pltpu.VMEM((_BPW, D), jnp.float32),
            pltpu.SemaphoreType.DMA,
        ],
    )()
    return k(tok_table, pos_table, word_ids, position_ids)


def _ln(x, g, b, eps=1e-5):
    m = jnp.mean(x, axis=-1, keepdims=True)
    v = jnp.mean((x - m) ** 2, axis=-1, keepdims=True)
    return (x - m) / jnp.sqrt(v + eps) * g + b


def _gat_kernel(h0_ref, adj_ref, et_ref, ek_ref, wq_ref, wk_ref, wv_ref,
                wo_ref, ln1g_ref, ln1b_ref, w1_ref, b1_ref, w2_ref, b2_ref,
                ln2g_ref, ln2b_ref, cls_ref, wc_ref, bc_ref,
                out_ref, h_scr, k_scr, v_scr):
    l = pl.program_id(0)
    i = pl.program_id(1)
    scale = jnp.float32(1.0 / (DH ** 0.5))

    @pl.when((l == 0) & (i == 0))
    def _():
        h_scr[...] = h0_ref[...]

    @pl.when(i == 0)
    def _():
        hf = h_scr[...]
        k_scr[...] = jnp.dot(hf, wk_ref[...].reshape(D, D),
                             preferred_element_type=jnp.float32)
        v_scr[...] = jnp.dot(hf, wv_ref[...].reshape(D, D),
                             preferred_element_type=jnp.float32)

    hb = h_scr[pl.ds(i * BN, BN), :]
    q = jnp.dot(hb, wq_ref[...].reshape(D, D),
                preferred_element_type=jnp.float32)
    k = k_scr[...]
    v = v_scr[...]
    ek = ek_ref[...]
    adjb = adj_ref[...]
    etb = et_ref[...]

    amask = jnp.where(adjb > 0, 0.0, jnp.float32(-1e9))
    tmasks = [(etb == t).astype(jnp.bfloat16) for t in range(NET)]

    ctx_parts = []
    for hh in range(H):
        sl = slice(hh * DH, (hh + 1) * DH)
        qh = q[:, sl]
        kh = k[:, sl]
        s = jax.lax.dot_general(
            qh, kh, (((1,), (1,)), ((), ())),
            preferred_element_type=jnp.float32) * scale
        qe = (jax.lax.dot_general(
            qh, ek[:, sl], (((1,), (1,)), ((), ())),
            preferred_element_type=jnp.float32) * scale
            ).astype(jnp.bfloat16)
        bias = tmasks[0] * qe[:, 0:1]
        for t in range(1, NET):
            bias = bias + tmasks[t] * qe[:, t:t + 1]
        s = s + bias.astype(jnp.float32) + amask
        m = jnp.max(s, axis=1, keepdims=True)
        p = jnp.exp(s - m)
        denom = jnp.sum(p, axis=1, keepdims=True)
        ctx_parts.append(jax.lax.dot_general(
            p, v[:, sl], (((1,), (0,)), ((), ())),
            preferred_element_type=jnp.float32) / denom)
    ctx = jnp.concatenate(ctx_parts, axis=1)

    h1 = hb + jnp.dot(ctx, wo_ref[...].reshape(D, D),
                      preferred_element_type=jnp.float32)
    h1 = _ln(h1, ln1g_ref[...].reshape(1, D), ln1b_ref[...].reshape(1, D))
    ffv = jnp.maximum(
        jnp.dot(h1, w1_ref[...].reshape(D, FF),
                preferred_element_type=jnp.float32)
        + b1_ref[...].reshape(1, FF), 0.0)
    ffv = jnp.dot(ffv, w2_ref[...].reshape(FF, D),
                  preferred_element_type=jnp.float32) + b2_ref[...].reshape(1, D)
    hn = _ln(h1 + ffv, ln2g_ref[...].reshape(1, D), ln2b_ref[...].reshape(1, D))
    h_scr[pl.ds(i * BN, BN), :] = hn

    @pl.when((l == L - 1) & (i == NQB - 1))
    def _():
        hf = h_scr[...]
        idx = cls_ref[...]
        iota = jax.lax.broadcasted_iota(jnp.int32, (B, N), 1)
        oh = (iota == idx).astype(jnp.float32)
        clsh = jnp.dot(oh, hf, preferred_element_type=jnp.float32)
        out_ref[...] = jnp.dot(clsh, wc_ref[...],
                               preferred_element_type=jnp.float32) + bc_ref[...]


@jax.jit
def _gat_forward(h0, adj, edge_types, edge_table, Wq, Wk, Wv, Wo,
                 ln1_g, ln1_b, W1, b1, W2, b2, ln2_g, ln2_b,
                 cls_node2d, Wc, bc2d):
    grid = (L, NQB)
    full = lambda shape: pl.BlockSpec(shape, lambda l, i: (0,) * len(shape))
    per_l = lambda shape: pl.BlockSpec((1,) + shape, lambda l, i: (l,) + (0,) * len(shape))
    row_block = pl.BlockSpec((BN, N), lambda l, i: (i, 0))
    return pl.pallas_call(
        _gat_kernel,
        grid=grid,
        in_specs=[
            full((N, D)),
            row_block,
            row_block,
            full((NET, D)),
            per_l((D, D)),
            per_l((D, D)),
            per_l((D, D)),
            per_l((D, D)),
            per_l((1, D)),
            per_l((1, D)),
            per_l((D, FF)),
            per_l((1, FF)),
            per_l((FF, D)),
            per_l((1, D)),
            per_l((1, D)),
            per_l((1, D)),
            full((B, 1)),
            full((D, NCLASS)),
            full((1, NCLASS)),
        ],
        out_specs=pl.BlockSpec((B, NCLASS), lambda l, i: (0, 0)),
        out_shape=jax.ShapeDtypeStruct((B, NCLASS), jnp.float32),
        scratch_shapes=[
            pltpu.VMEM((N, D), jnp.float32),
            pltpu.VMEM((N, D), jnp.float32),
            pltpu.VMEM((N, D), jnp.float32),
        ],
    )(h0, adj, edge_types, edge_table, Wq, Wk, Wv, Wo,
      ln1_g, ln1_b, W1, b1, W2, b2, ln2_g, ln2_b,
      cls_node2d, Wc, bc2d)


def kernel(word_ids, position_ids, adj, edge_types, cls_node, tok_table,
           pos_table, edge_table, Wq, Wk, Wv, Wo, ln1_g, ln1_b, W1, b1,
           W2, b2, ln2_g, ln2_b, Wc, bc):
    h0 = _embed_sc(tok_table, pos_table,
                   word_ids.astype(jnp.int32), position_ids.astype(jnp.int32))
    return _gat_forward(
        h0, adj, edge_types, edge_table, Wq, Wk, Wv, Wo,
        ln1_g.reshape(L, 1, D), ln1_b.reshape(L, 1, D),
        W1, b1.reshape(L, 1, FF), W2, b2.reshape(L, 1, D),
        ln2_g.reshape(L, 1, D), ln2_b.reshape(L, 1, D),
        cls_node.astype(jnp.int32).reshape(B, 1), Wc, bc.reshape(1, NCLASS))


# speedup vs baseline: 2157.6538x; 1.1230x over previous
import functools

import jax
import jax.numpy as jnp
from jax import lax
from jax.experimental import pallas as pl
from jax.experimental.pallas import tpu as pltpu
from jax.experimental.pallas import tpu_sc as plsc

N = 2048
D = 128
H = 8
DH = 16
L = 2
FF = 512
NET = 17
NCLASS = 5
B = 16

BN = 256
NQB = N // BN


_NC, _NS, _LANES = 2, 16, 16
_NW = _NC * _NS
_BPW = N // _NW


def _emb_body(tok_hbm, pos_hbm, wid_hbm, pid_hbm, out_hbm,
              idx_v, rows_a, rows_b, sem):
    wid = lax.axis_index("s") * _NC + lax.axis_index("c")
    base = wid * _BPW
    pltpu.sync_copy(wid_hbm.at[pl.ds(base, _BPW)], idx_v)
    pltpu.async_copy(tok_hbm.at[idx_v], rows_a, sem).wait()
    pltpu.sync_copy(pid_hbm.at[pl.ds(base, _BPW)], idx_v)
    pltpu.async_copy(pos_hbm.at[idx_v], rows_b, sem).wait()

    def body(r, carry):
        for c in range(D // _LANES):
            sl = pl.ds(c * _LANES, _LANES)
            rows_a[r, sl] = rows_a[r, sl] + rows_b[r, sl]
        return carry

    lax.fori_loop(0, _BPW, body, 0)
    pltpu.sync_copy(rows_a, out_hbm.at[pl.ds(base, _BPW)])


def _embed_sc(tok_table, pos_table, word_ids, position_ids):
    k = functools.partial(
        pl.kernel, _emb_body,
        mesh=plsc.VectorSubcoreMesh(core_axis_name="c", subcore_axis_name="s"),
        out_type=jax.ShapeDtypeStruct((N, D), jnp.float32),
        scratch_types=[
            pltpu.VMEM((_BPW,), jnp.int32),
            pltpu.VMEM((_BPW, D), jnp.float32),
            pltpu.VMEM((_BPW, D), jnp.float32),
            pltpu.SemaphoreType.DMA,
        ],
    )()
    return k(tok_table, pos_table, word_ids, position_ids)


def _ln(x, g, b, eps=1e-5):
    m = jnp.mean(x, axis=-1, keepdims=True)
    v = jnp.mean((x - m) ** 2, axis=-1, keepdims=True)
    return (x - m) / jnp.sqrt(v + eps) * g + b


def _gat_kernel(h0_ref, adj_ref, et_ref, ek_ref, wq_ref, wk_ref, wv_ref,
                wo_ref, ln1g_ref, ln1b_ref, w1_ref, b1_ref, w2_ref, b2_ref,
                ln2g_ref, ln2b_ref, cls_ref, wc_ref, bc_ref,
                out_ref, h_scr, k_scr, v_scr):
    l = pl.program_id(0)
    i = pl.program_id(1)
    scale = jnp.float32(1.0 / (DH ** 0.5))

    @pl.when((l == 0) & (i == 0))
    def _():
        h_scr[...] = h0_ref[...]

    @pl.when(i == 0)
    def _():
        hf = h_scr[...]
        k_scr[...] = jnp.dot(hf, wk_ref[...].reshape(D, D),
                             preferred_element_type=jnp.float32)
        v_scr[...] = jnp.dot(hf, wv_ref[...].reshape(D, D),
                             preferred_element_type=jnp.float32)

    hb = h_scr[pl.ds(i * BN, BN), :]
    q = jnp.dot(hb, wq_ref[...].reshape(D, D),
                preferred_element_type=jnp.float32)
    k = k_scr[...]
    v = v_scr[...]
    ek = ek_ref[...]
    adjb = adj_ref[...]
    etb = et_ref[...]

    amask = jnp.where(adjb > 0, 0.0, jnp.float32(-1e9))
    tmasks = [(etb == t).astype(jnp.bfloat16) for t in range(NET)]

    ctx_parts = []
    for hh in range(H):
        sl = slice(hh * DH, (hh + 1) * DH)
        qh = q[:, sl]
        kh = k[:, sl]
        s = jax.lax.dot_general(
            qh, kh, (((1,), (1,)), ((), ())),
            preferred_element_type=jnp.float32) * scale
        qe = (jax.lax.dot_general(
            qh, ek[:, sl], (((1,), (1,)), ((), ())),
            preferred_element_type=jnp.float32) * scale
            ).astype(jnp.bfloat16)
        bias = tmasks[0] * qe[:, 0:1]
        for t in range(1, NET):
            bias = bias + tmasks[t] * qe[:, t:t + 1]
        s = s + bias.astype(jnp.float32) + amask
        m = jnp.max(s, axis=1, keepdims=True)
        p = jnp.exp(s - m)
        denom = jnp.sum(p, axis=1, keepdims=True)
        ctx_parts.append(jax.lax.dot_general(
            p, v[:, sl], (((1,), (0,)), ((), ())),
            preferred_element_type=jnp.float32) / denom)
    ctx = jnp.concatenate(ctx_parts, axis=1)

    h1 = hb + jnp.dot(ctx, wo_ref[...].reshape(D, D),
                      preferred_element_type=jnp.float32)
    h1 = _ln(h1, ln1g_ref[...].reshape(1, D), ln1b_ref[...].reshape(1, D))
    ffv = jnp.maximum(
        jnp.dot(h1, w1_ref[...].reshape(D, FF),
                preferred_element_type=jnp.float32)
        + b1_ref[...].reshape(1, FF), 0.0)
    ffv = jnp.dot(ffv, w2_ref[...].reshape(FF, D),
                  preferred_element_type=jnp.float32) + b2_ref[...].reshape(1, D)
    hn = _ln(h1 + ffv, ln2g_ref[...].reshape(1, D), ln2b_ref[...].reshape(1, D))
    h_scr[pl.ds(i * BN, BN), :] = hn

    @pl.when((l == L - 1) & (i == NQB - 1))
    def _():
        hf = h_scr[...]
        idx = cls_ref[...]
        iota = jax.lax.broadcasted_iota(jnp.int32, (B, N), 1)
        oh = (iota == idx).astype(jnp.float32)
        clsh = jnp.dot(oh, hf, preferred_element_type=jnp.float32)
        out_ref[...] = jnp.dot(clsh, wc_ref[...],
                               preferred_element_type=jnp.float32) + bc_ref[...]


@jax.jit
def _gat_forward(h0, adj, edge_types, edge_table, Wq, Wk, Wv, Wo,
                 ln1_g, ln1_b, W1, b1, W2, b2, ln2_g, ln2_b,
                 cls_node2d, Wc, bc2d):
    grid = (L, NQB)
    full = lambda shape: pl.BlockSpec(shape, lambda l, i: (0,) * len(shape))
    per_l = lambda shape: pl.BlockSpec((1,) + shape, lambda l, i: (l,) + (0,) * len(shape))
    row_block = pl.BlockSpec((BN, N), lambda l, i: (i, 0))
    return pl.pallas_call(
        _gat_kernel,
        grid=grid,
        in_specs=[
            full((N, D)),
            row_block,
            row_block,
            full((NET, D)),
            per_l((D, D)),
            per_l((D, D)),
            per_l((D, D)),
            per_l((D, D)),
            per_l((1, D)),
            per_l((1, D)),
            per_l((D, FF)),
            per_l((1, FF)),
            per_l((FF, D)),
            per_l((1, D)),
            per_l((1, D)),
            per_l((1, D)),
            full((B, 1)),
            full((D, NCLASS)),
            full((1, NCLASS)),
        ],
        out_specs=pl.BlockSpec((B, NCLASS), lambda l, i: (0, 0)),
        out_shape=jax.ShapeDtypeStruct((B, NCLASS), jnp.float32),
        scratch_shapes=[
            pltpu.VMEM((N, D), jnp.float32),
            pltpu.VMEM((N, D), jnp.float32),
            pltpu.VMEM((N, D), jnp.float32),
        ],
    )(h0, adj, edge_types, edge_table, Wq, Wk, Wv, Wo,
      ln1_g, ln1_b, W1, b1, W2, b2, ln2_g, ln2_b,
      cls_node2d, Wc, bc2d)


def kernel(word_ids, position_ids, adj, edge_types, cls_node, tok_table,
           pos_table, edge_table, Wq, Wk, Wv, Wo, ln1_g, ln1_b, W1, b1,
           W2, b2, ln2_g, ln2_b, Wc, bc):
    h0 = _embed_sc(tok_table, pos_table,
                   word_ids.astype(jnp.int32), position_ids.astype(jnp.int32))
    return _gat_forward(
        h0, adj, edge_types, edge_table, Wq, Wk, Wv, Wo,
        ln1_g.reshape(L, 1, D), ln1_b.reshape(L, 1, D),
        W1, b1.reshape(L, 1, FF), W2, b2.reshape(L, 1, D),
        ln2_g.reshape(L, 1, D), ln2_b.reshape(L, 1, D),
        cls_node.astype(jnp.int32).reshape(B, 1), Wc, bc.reshape(1, NCLASS))
